# Initial kernel scaffold; baseline (speedup 1.0000x reference)
import jax
import jax.numpy as jnp
from jax.experimental import pallas as pl


def kernel(x, y, fp, distinct_xp, distinct_yp, grad_x_fp, grad_y_fp):
    raise NotImplementedError("write your pallas kernel here")


# speedup vs baseline: 357.7325x; 357.7325x over previous
import functools

import jax
import jax.numpy as jnp
from jax import lax
from jax.experimental import pallas as pl
from jax.experimental.pallas import tpu as pltpu
from jax.experimental.pallas import tpu_sc as plsc

NC = 2
NS = 16
NW = NC * NS
L = 16

K = 1024
ISL = 128
NSL = K // ISL


def _make_kernel(n, ny, nx, c):
    nq = n // NW
    nchunk = nq // K
    mesh = plsc.VectorSubcoreMesh(
        core_axis_name="c", subcore_axis_name="s", num_cores=NC, num_subcores=NS
    )

    @functools.partial(
        pl.kernel,
        out_type=jax.ShapeDtypeStruct((n, c), jnp.float32),
        mesh=mesh,
        scratch_types=[
            pltpu.VMEM((K,), jnp.float32),
            pltpu.VMEM((K,), jnp.float32),
            [pltpu.VMEM((NSL, ISL), jnp.int32) for _ in range(4)],
            [pltpu.VMEM((K,), jnp.float32) for _ in range(4)],
            [pltpu.VMEM((K, c), jnp.float32) for _ in range(4)],
            pltpu.VMEM((K, c), jnp.float32),
            pltpu.SemaphoreType.DMA,
        ],
        compiler_params=pltpu.CompilerParams(use_tc_tiling_on_sc=False),
    )
    def bilerp(x_hbm, y_hbm, tab_hbm, out_hbm, x_v, y_v, idx_v, w_v, f_v, o_v, sem):
        wid = lax.axis_index("s") * NC + lax.axis_index("c")
        qbase = wid * nq

        fxmax = jnp.float32(nx - 1)
        fymax = jnp.float32(ny - 1)

        def chunk_body(ch, _):
            base = qbase + ch * K
            pltpu.sync_copy(x_hbm.at[pl.ds(base, K)], x_v)
            pltpu.sync_copy(y_hbm.at[pl.ds(base, K)], y_v)

            def ix_body(i, _):
                xs = x_v[pl.ds(i * L, L)]
                ys = y_v[pl.ds(i * L, L)]
                fx = xs * fxmax
                fy = ys * fymax
                cfx = jnp.minimum(jnp.maximum(fx, 0.0), fxmax - 1.0)
                cfy = jnp.minimum(jnp.maximum(fy, 0.0), fymax - 1.0)
                ix = cfx.astype(jnp.int32)
                iy = cfy.astype(jnp.int32)
                tx = fx - ix.astype(jnp.float32)
                ty = fy - iy.astype(jnp.float32)
                inr = jnp.where(
                    (xs >= 0.0) & (xs <= 1.0) & (ys >= 0.0) & (ys <= 1.0),
                    jnp.full((L,), 1.0, jnp.float32),
                    jnp.full((L,), 0.0, jnp.float32),
                )
                ux = inr - tx * inr
                uy = 1.0 - ty
                w_v[0][pl.ds(i * L, L)] = uy * ux
                w_v[1][pl.ds(i * L, L)] = uy * (tx * inr)
                w_v[2][pl.ds(i * L, L)] = ty * ux
                w_v[3][pl.ds(i * L, L)] = ty * (tx * inr)
                i00 = iy * nx + ix
                row = i * L // ISL
                col = (i * L) % ISL
                idx_v[0][row, pl.ds(col, L)] = i00
                idx_v[1][row, pl.ds(col, L)] = i00 + 1
                idx_v[2][row, pl.ds(col, L)] = i00 + nx
                idx_v[3][row, pl.ds(col, L)] = i00 + nx + 1
                return _

            lax.fori_loop(0, K // L, ix_body, None)

            cps = []
            for k in range(4):
                for j in range(NSL):
                    cps.append(
                        pltpu.async_copy(
                            tab_hbm.at[idx_v[k].at[j]],
                            f_v[k].at[pl.ds(j * ISL, ISL)],
                            sem,
                        )
                    )
            for cp in cps:
                cp.wait()

            def cb_body(i, _):
                w0 = w_v[0][pl.ds(i * L, L)]
                w1 = w_v[1][pl.ds(i * L, L)]
                w2 = w_v[2][pl.ds(i * L, L)]
                w3 = w_v[3][pl.ds(i * L, L)]
                for qq in range(L):
                    q = i * L + qq
                    o_v[q] = (
                        jnp.broadcast_to(w0[qq], (L,)) * f_v[0][q]
                        + jnp.broadcast_to(w1[qq], (L,)) * f_v[1][q]
                        + jnp.broadcast_to(w2[qq], (L,)) * f_v[2][q]
                        + jnp.broadcast_to(w3[qq], (L,)) * f_v[3][q]
                    )
                return _

            lax.fori_loop(0, K // L, cb_body, None)

            pltpu.sync_copy(o_v, out_hbm.at[pl.ds(base, K)])
            return _

        lax.fori_loop(0, nchunk, chunk_body, None)

    return bilerp


def kernel(x, y, fp, distinct_xp, distinct_yp, grad_x_fp, grad_y_fp):
    x = jnp.ravel(x)
    y = jnp.ravel(y)
    ny, nx, c = fp.shape
    n = x.shape[0]
    assert n % (NW * K) == 0
    tab = fp.reshape(ny * nx, c)
    return _make_kernel(n, ny, nx, c)(x, y, tab)


# speedup vs baseline: 405.7087x; 1.1341x over previous
import functools

import jax
import jax.numpy as jnp
from jax import lax
from jax.experimental import pallas as pl
from jax.experimental.pallas import tpu as pltpu
from jax.experimental.pallas import tpu_sc as plsc

NC = 2
NS = 16
NW = NC * NS
L = 16

K = 512
ISL = 128
NSL = K // ISL


def _make_kernel(n, ny, nx, c):
    nq = n // NW
    nchunk = nq // K
    assert nchunk % 2 == 0
    mesh = plsc.VectorSubcoreMesh(
        core_axis_name="c", subcore_axis_name="s", num_cores=NC, num_subcores=NS
    )

    def buf():
        return {
            "x": pltpu.VMEM((K,), jnp.float32),
            "y": pltpu.VMEM((K,), jnp.float32),
            "idx": [pltpu.VMEM((NSL, ISL), jnp.int32) for _ in range(4)],
            "w": [pltpu.VMEM((K,), jnp.float32) for _ in range(4)],
            "f": [pltpu.VMEM((K, c), jnp.float32) for _ in range(4)],
            "o": pltpu.VMEM((K, c), jnp.float32),
            "sem": pltpu.SemaphoreType.DMA,
        }

    @functools.partial(
        pl.kernel,
        out_type=jax.ShapeDtypeStruct((n, c), jnp.float32),
        mesh=mesh,
        scratch_types=[buf(), buf()],
        compiler_params=pltpu.CompilerParams(use_tc_tiling_on_sc=False),
    )
    def bilerp(x_hbm, y_hbm, tab_hbm, out_hbm, b0, b1):
        bufs = (b0, b1)
        wid = lax.axis_index("s") * NC + lax.axis_index("c")
        qbase = wid * nq

        fxmax = jnp.float32(nx - 1)
        fymax = jnp.float32(ny - 1)

        def gathers(b):
            return [
                (tab_hbm.at[b["idx"][k].at[j]],
                 b["f"][k].at[pl.ds(j * ISL, ISL)])
                for k in range(4)
                for j in range(NSL)
            ]

        def fire(ch, p):
            b = bufs[p]
            base = qbase + ch * K
            pltpu.sync_copy(x_hbm.at[pl.ds(base, K)], b["x"])
            pltpu.sync_copy(y_hbm.at[pl.ds(base, K)], b["y"])

            def ix_body(i, _):
                xs = b["x"][pl.ds(i * L, L)]
                ys = b["y"][pl.ds(i * L, L)]
                fx = xs * fxmax
                fy = ys * fymax
                cfx = jnp.minimum(jnp.maximum(fx, 0.0), fxmax - 1.0)
                cfy = jnp.minimum(jnp.maximum(fy, 0.0), fymax - 1.0)
                ix = cfx.astype(jnp.int32)
                iy = cfy.astype(jnp.int32)
                tx = fx - ix.astype(jnp.float32)
                ty = fy - iy.astype(jnp.float32)
                inr = jnp.where(
                    (xs >= 0.0) & (xs <= 1.0) & (ys >= 0.0) & (ys <= 1.0),
                    jnp.full((L,), 1.0, jnp.float32),
                    jnp.full((L,), 0.0, jnp.float32),
                )
                txm = tx * inr
                ux = inr - txm
                uy = 1.0 - ty
                b["w"][0][pl.ds(i * L, L)] = uy * ux
                b["w"][1][pl.ds(i * L, L)] = uy * txm
                b["w"][2][pl.ds(i * L, L)] = ty * ux
                b["w"][3][pl.ds(i * L, L)] = ty * txm
                i00 = iy * nx + ix
                row = i * L // ISL
                col = (i * L) % ISL
                b["idx"][0][row, pl.ds(col, L)] = i00
                b["idx"][1][row, pl.ds(col, L)] = i00 + 1
                b["idx"][2][row, pl.ds(col, L)] = i00 + nx
                b["idx"][3][row, pl.ds(col, L)] = i00 + nx + 1
                return _

            lax.fori_loop(0, K // L, ix_body, None)
            for src, dst in gathers(b):
                pltpu.async_copy(src, dst, b["sem"])

        def consume(ch, p):
            b = bufs[p]
            base = qbase + ch * K
            for src, dst in gathers(b):
                pltpu.make_async_copy(src, dst, b["sem"]).wait()

            def cb_body(i, _):
                w0 = b["w"][0][pl.ds(i * L, L)]
                w1 = b["w"][1][pl.ds(i * L, L)]
                w2 = b["w"][2][pl.ds(i * L, L)]
                w3 = b["w"][3][pl.ds(i * L, L)]
                for qq in range(L):
                    q = i * L + qq
                    b["o"][q] = (
                        jnp.broadcast_to(w0[qq], (L,)) * b["f"][0][q]
                        + jnp.broadcast_to(w1[qq], (L,)) * b["f"][1][q]
                        + jnp.broadcast_to(w2[qq], (L,)) * b["f"][2][q]
                        + jnp.broadcast_to(w3[qq], (L,)) * b["f"][3][q]
                    )
                return _

            lax.fori_loop(0, K // L, cb_body, None)
            pltpu.sync_copy(b["o"], out_hbm.at[pl.ds(base, K)])

        fire(0, 0)

        def pair_body(j, _):
            a = 2 * j
            fire(a + 1, 1)
            consume(a, 0)
            fire(a + 2, 0)
            consume(a + 1, 1)
            return _

        lax.fori_loop(0, nchunk // 2 - 1, pair_body, None)
        fire(nchunk - 1, 1)
        consume(nchunk - 2, 0)
        consume(nchunk - 1, 1)

    return bilerp


def kernel(x, y, fp, distinct_xp, distinct_yp, grad_x_fp, grad_y_fp):
    x = jnp.ravel(x)
    y = jnp.ravel(y)
    ny, nx, c = fp.shape
    n = x.shape[0]
    assert n % (NW * K) == 0
    tab = fp.reshape(ny * nx, c)
    return _make_kernel(n, ny, nx, c)(x, y, tab)


# speedup vs baseline: 432.6278x; 1.0664x over previous
import functools

import jax
import jax.numpy as jnp
from jax import lax
from jax.experimental import pallas as pl
from jax.experimental.pallas import tpu as pltpu
from jax.experimental.pallas import tpu_sc as plsc

NC = 2
NS = 16
NW = NC * NS
L = 16

K = 512
ISL = 128
NSL = K // ISL


def _make_kernel(n, ny, nx, c):
    nq = n // NW
    nchunk = nq // K
    assert nchunk % 2 == 0 and nchunk >= 4
    mesh = plsc.VectorSubcoreMesh(
        core_axis_name="c", subcore_axis_name="s", num_cores=NC, num_subcores=NS
    )

    def buf():
        return {
            "x": pltpu.VMEM((K,), jnp.float32),
            "y": pltpu.VMEM((K,), jnp.float32),
            "idx": [pltpu.VMEM((NSL, ISL), jnp.int32) for _ in range(4)],
            "w": [pltpu.VMEM((K,), jnp.float32) for _ in range(4)],
            "f": [pltpu.VMEM((K, c), jnp.float32) for _ in range(4)],
            "o": pltpu.VMEM((K * c,), jnp.float32),
            "gsem": pltpu.SemaphoreType.DMA,
            "xysem": pltpu.SemaphoreType.DMA,
            "osem": pltpu.SemaphoreType.DMA,
        }

    @functools.partial(
        pl.kernel,
        out_type=jax.ShapeDtypeStruct((n * c,), jnp.float32),
        mesh=mesh,
        scratch_types=[buf(), buf()],
        compiler_params=pltpu.CompilerParams(use_tc_tiling_on_sc=False),
    )
    def bilerp(x_hbm, y_hbm, tab_hbm, out_hbm, b0, b1):
        bufs = (b0, b1)
        wid = lax.axis_index("s") * NC + lax.axis_index("c")
        qbase = wid * nq

        fxmax = jnp.float32(nx - 1)
        fymax = jnp.float32(ny - 1)

        def gathers(b):
            return [
                (tab_hbm.at[b["idx"][k].at[j]],
                 b["f"][k].at[pl.ds(j * ISL, ISL)])
                for k in range(4)
                for j in range(NSL)
            ]

        def xy_copies(ch, p):
            b = bufs[p]
            base = qbase + ch * K
            return [
                (x_hbm.at[pl.ds(base, K)], b["x"]),
                (y_hbm.at[pl.ds(base, K)], b["y"]),
            ]

        def prefetch_xy(ch, p):
            for src, dst in xy_copies(ch, p):
                pltpu.async_copy(src, dst, bufs[p]["xysem"])

        def fire(ch, p, prefetch_next=True):
            b = bufs[p]
            for src, dst in xy_copies(ch, p):
                pltpu.make_async_copy(src, dst, b["xysem"]).wait()
            if prefetch_next:
                prefetch_xy(ch + 1, 1 - p)

            def ix_body(i, _):
                xs = b["x"][pl.ds(i * L, L)]
                ys = b["y"][pl.ds(i * L, L)]
                fx = xs * fxmax
                fy = ys * fymax
                cfx = jnp.minimum(jnp.maximum(fx, 0.0), fxmax - 1.0)
                cfy = jnp.minimum(jnp.maximum(fy, 0.0), fymax - 1.0)
                ix = cfx.astype(jnp.int32)
                iy = cfy.astype(jnp.int32)
                tx = fx - ix.astype(jnp.float32)
                ty = fy - iy.astype(jnp.float32)
                inr = jnp.where(
                    (xs >= 0.0) & (xs <= 1.0) & (ys >= 0.0) & (ys <= 1.0),
                    jnp.full((L,), 1.0, jnp.float32),
                    jnp.full((L,), 0.0, jnp.float32),
                )
                txm = tx * inr
                ux = inr - txm
                uy = 1.0 - ty
                b["w"][0][pl.ds(i * L, L)] = uy * ux
                b["w"][1][pl.ds(i * L, L)] = uy * txm
                b["w"][2][pl.ds(i * L, L)] = ty * ux
                b["w"][3][pl.ds(i * L, L)] = ty * txm
                i00 = iy * nx + ix
                row = i * L // ISL
                col = (i * L) % ISL
                b["idx"][0][row, pl.ds(col, L)] = i00
                b["idx"][1][row, pl.ds(col, L)] = i00 + 1
                b["idx"][2][row, pl.ds(col, L)] = i00 + nx
                b["idx"][3][row, pl.ds(col, L)] = i00 + nx + 1
                return _

            lax.fori_loop(0, K // L, ix_body, None)
            for src, dst in gathers(b):
                pltpu.async_copy(src, dst, b["gsem"])

        def out_copy(ch, p):
            base = qbase + ch * K
            return bufs[p]["o"], out_hbm.at[pl.ds(base * c, K * c)]

        def drain_out(ch, p):
            src, dst = out_copy(ch, p)
            pltpu.make_async_copy(src, dst, bufs[p]["osem"]).wait()

        def consume(ch, p, drain_prev=True):
            b = bufs[p]
            for src, dst in gathers(b):
                pltpu.make_async_copy(src, dst, b["gsem"]).wait()
            if drain_prev:
                drain_out(ch - 2, p)

            def cb_body(i, _):
                w0 = b["w"][0][pl.ds(i * L, L)]
                w1 = b["w"][1][pl.ds(i * L, L)]
                w2 = b["w"][2][pl.ds(i * L, L)]
                w3 = b["w"][3][pl.ds(i * L, L)]
                for qq in range(L):
                    q = i * L + qq
                    b["o"][pl.ds(q * c, c)] = (
                        jnp.broadcast_to(w0[qq], (L,)) * b["f"][0][q]
                        + jnp.broadcast_to(w1[qq], (L,)) * b["f"][1][q]
                        + jnp.broadcast_to(w2[qq], (L,)) * b["f"][2][q]
                        + jnp.broadcast_to(w3[qq], (L,)) * b["f"][3][q]
                    )
                return _

            lax.fori_loop(0, K // L, cb_body, None)
            src, dst = out_copy(ch, p)
            pltpu.async_copy(src, dst, b["osem"])

        prefetch_xy(0, 0)
        fire(0, 0)
        fire(1, 1)
        consume(0, 0, drain_prev=False)
        fire(2, 0)
        consume(1, 1, drain_prev=False)

        def pair_body(j, _):
            a = 2 * j
            fire(a + 3, 1)
            consume(a + 2, 0)
            fire(a + 4, 0)
            consume(a + 3, 1)
            return _

        lax.fori_loop(0, nchunk // 2 - 2, pair_body, None)
        fire(nchunk - 1, 1, prefetch_next=False)
        consume(nchunk - 2, 0)
        consume(nchunk - 1, 1)
        drain_out(nchunk - 2, 0)
        drain_out(nchunk - 1, 1)

    return bilerp


def kernel(x, y, fp, distinct_xp, distinct_yp, grad_x_fp, grad_y_fp):
    x = jnp.ravel(x)
    y = jnp.ravel(y)
    ny, nx, c = fp.shape
    n = x.shape[0]
    assert n % (NW * K) == 0
    tab = fp.reshape(ny * nx, c)
    out = _make_kernel(n, ny, nx, c)(x, y, tab)
    return out.reshape(n, c)


# speedup vs baseline: 1049.0892x; 2.4249x over previous
import functools

import jax
import jax.numpy as jnp
from jax import lax
from jax.experimental import pallas as pl
from jax.experimental.pallas import tpu as pltpu
from jax.experimental.pallas import tpu_sc as plsc

NC = 2
NS = 16
NW = NC * NS
L = 16

K = 512
ISL = 128
NSL = K // ISL


def _make_kernel(n, ny, nx, c):
    nq = n // NW
    nchunk = nq // K
    assert nchunk % 2 == 0 and nchunk >= 4
    mesh = plsc.VectorSubcoreMesh(
        core_axis_name="c", subcore_axis_name="s", num_cores=NC, num_subcores=NS
    )

    def buf():
        return {
            "x": pltpu.VMEM((K,), jnp.float32),
            "y": pltpu.VMEM((K,), jnp.float32),
            "idx": [pltpu.VMEM((NSL, ISL), jnp.int32) for _ in range(4)],
            "w": [pltpu.VMEM((K,), jnp.float32) for _ in range(4)],
            "f": [pltpu.VMEM((K, c), jnp.float32) for _ in range(4)],
            "o": pltpu.VMEM((2, K * c // 2), jnp.float32),
            "gsem": pltpu.SemaphoreType.DMA,
            "xysem": pltpu.SemaphoreType.DMA,
            "osem": pltpu.SemaphoreType.DMA,
        }

    @functools.partial(
        pl.kernel,
        out_type=jax.ShapeDtypeStruct((n * c,), jnp.float32),
        mesh=mesh,
        scratch_types=[buf(), buf()],
        compiler_params=pltpu.CompilerParams(use_tc_tiling_on_sc=False),
    )
    def bilerp(x_hbm, y_hbm, tab_hbm, out_hbm, b0, b1):
        bufs = (b0, b1)
        wid = lax.axis_index("s") * NC + lax.axis_index("c")
        qbase = wid * nq

        fxmax = jnp.float32(nx - 1)
        fymax = jnp.float32(ny - 1)

        def gathers(b):
            return [
                (tab_hbm.at[b["idx"][k].at[j]],
                 b["f"][k].at[pl.ds(j * ISL, ISL)])
                for k in range(4)
                for j in range(NSL)
            ]

        def xy_copies(ch, p):
            b = bufs[p]
            base = qbase + ch * K
            return [
                (x_hbm.at[pl.ds(base, K)], b["x"]),
                (y_hbm.at[pl.ds(base, K)], b["y"]),
            ]

        def prefetch_xy(ch, p):
            for src, dst in xy_copies(ch, p):
                pltpu.async_copy(src, dst, bufs[p]["xysem"])

        def fire(ch, p, prefetch_next=True):
            b = bufs[p]
            for src, dst in xy_copies(ch, p):
                pltpu.make_async_copy(src, dst, b["xysem"]).wait()
            if prefetch_next:
                prefetch_xy(ch + 1, 1 - p)

            def ix_body(i, _):
                xs = b["x"][pl.ds(i * L, L)]
                ys = b["y"][pl.ds(i * L, L)]
                fx = xs * fxmax
                fy = ys * fymax
                cfx = jnp.minimum(jnp.maximum(fx, 0.0), fxmax - 1.0)
                cfy = jnp.minimum(jnp.maximum(fy, 0.0), fymax - 1.0)
                ix = cfx.astype(jnp.int32)
                iy = cfy.astype(jnp.int32)
                tx = fx - ix.astype(jnp.float32)
                ty = fy - iy.astype(jnp.float32)
                inr = jnp.where(
                    (xs >= 0.0) & (xs <= 1.0) & (ys >= 0.0) & (ys <= 1.0),
                    jnp.full((L,), 1.0, jnp.float32),
                    jnp.full((L,), 0.0, jnp.float32),
                )
                txm = tx * inr
                ux = inr - txm
                uy = 1.0 - ty
                b["w"][0][pl.ds(i * L, L)] = uy * ux
                b["w"][1][pl.ds(i * L, L)] = uy * txm
                b["w"][2][pl.ds(i * L, L)] = ty * ux
                b["w"][3][pl.ds(i * L, L)] = ty * txm
                i00 = iy * nx + ix
                row = i * L // ISL
                col = (i * L) % ISL
                b["idx"][0][row, pl.ds(col, L)] = i00
                b["idx"][1][row, pl.ds(col, L)] = i00 + 1
                b["idx"][2][row, pl.ds(col, L)] = i00 + nx
                b["idx"][3][row, pl.ds(col, L)] = i00 + nx + 1
                return _

            lax.fori_loop(0, K // L, ix_body, None)
            for src, dst in gathers(b):
                pltpu.async_copy(src, dst, b["gsem"])

        def out_copies(ch, p):
            base = qbase + ch * K
            half = K * c // 2
            return [
                (bufs[p]["o"].at[i],
                 out_hbm.at[pl.ds(i * (n * 8) + base * 8, half)])
                for i in range(2)
            ]

        def drain_out(ch, p):
            for src, dst in out_copies(ch, p):
                pltpu.make_async_copy(src, dst, bufs[p]["osem"]).wait()

        def consume(ch, p, drain_prev=True):
            b = bufs[p]
            for src, dst in gathers(b):
                pltpu.make_async_copy(src, dst, b["gsem"]).wait()
            if drain_prev:
                drain_out(ch - 2, p)

            lane = lax.iota(jnp.int32, L)

            def cb_outer(j2, _):
                def cb_inner(g2, _2):
                    qoff = j2 * 128 + g2 * L
                    w0 = b["w"][0][pl.ds(qoff, L)]
                    w1 = b["w"][1][pl.ds(qoff, L)]
                    w2 = b["w"][2][pl.ds(qoff, L)]
                    w3 = b["w"][3][pl.ds(qoff, L)]
                    u = []
                    for qq in range(L):
                        u.append(
                            jnp.broadcast_to(w0[qq], (L,))
                            * b["f"][0][qoff + qq]
                            + jnp.broadcast_to(w1[qq], (L,))
                            * b["f"][1][qoff + qq]
                            + jnp.broadcast_to(w2[qq], (L,))
                            * b["f"][2][qoff + qq]
                            + jnp.broadcast_to(w3[qq], (L,))
                            * b["f"][3][qoff + qq]
                        )
                    v = u
                    for s in (1, 2, 4, 8):
                        mask = (lane & s) == 0
                        perm = lane ^ s
                        nv = [None] * L
                        for k in range(L):
                            if k & s == 0:
                                nv[k] = jnp.where(mask, v[k], v[k | s][perm])
                            else:
                                nv[k] = jnp.where(mask, v[k & ~s][perm], v[k])
                        v = nv
                    for cc in range(c):
                        ih = cc // (c // 2)
                        r = cc % (c // 2)
                        b["o"][
                            ih,
                            pl.ds(j2 * 1024 + r * 128 + g2 * L, L),
                        ] = v[cc]
                    return _2

                return lax.fori_loop(0, 128 // L, cb_inner, None)

            lax.fori_loop(0, K // 128, cb_outer, None)
            for src, dst in out_copies(ch, p):
                pltpu.async_copy(src, dst, b["osem"])

        prefetch_xy(0, 0)
        fire(0, 0)
        fire(1, 1)
        consume(0, 0, drain_prev=False)
        fire(2, 0)
        consume(1, 1, drain_prev=False)

        def pair_body(j, _):
            a = 2 * j
            fire(a + 3, 1)
            consume(a + 2, 0)
            fire(a + 4, 0)
            consume(a + 3, 1)
            return _

        lax.fori_loop(0, nchunk // 2 - 2, pair_body, None)
        fire(nchunk - 1, 1, prefetch_next=False)
        consume(nchunk - 2, 0)
        consume(nchunk - 1, 1)
        drain_out(nchunk - 2, 0)
        drain_out(nchunk - 1, 1)

    return bilerp


def kernel(x, y, fp, distinct_xp, distinct_yp, grad_x_fp, grad_y_fp):
    x = jnp.ravel(x)
    y = jnp.ravel(y)
    ny, nx, c = fp.shape
    n = x.shape[0]
    assert n % (NW * K) == 0
    tab = fp.reshape(ny * nx, c)
    out = _make_kernel(n, ny, nx, c)(x, y, tab)
    return (
        out.reshape(2, n // 128, c // 2, 128)
        .transpose(1, 3, 0, 2)
        .reshape(n, c)
    )
